# baseline (speedup 1.0000x reference)
import functools

import jax
import jax.numpy as jnp
from jax import lax
from jax.experimental import pallas as pl
from jax.experimental.pallas import tpu as pltpu
from jax.experimental.pallas import tpu_sc as plsc

_TOPK = 8
_NEG = -1e30
_BIGI = 2**30


def _extract_top8(s, col, idx_base):
  ts, ti = [], []
  for _ in range(_TOPK):
    m = jnp.max(s, axis=1, keepdims=True)
    eq = s == m
    idx = jnp.min(jnp.where(eq, col, _BIGI), axis=1, keepdims=True)
    ts.append(m)
    ti.append(idx + idx_base)
    s = jnp.where(col == idx, _NEG, s)
  return jnp.concatenate(ts, axis=1), jnp.concatenate(ti, axis=1)


def _stage1_body(ndocs, dt, patient_ref, wpet_ref, bpe_ref, corpus_ref,
                 out_s_ref, out_i_ref, pe_ref):
  d = pl.program_id(1)
  bq = patient_ref.shape[0]

  @pl.when(d == 0)
  def _init():
    pe = jnp.dot(patient_ref[...], wpet_ref[...],
                 preferred_element_type=jnp.float32) + bpe_ref[...]
    n = jnp.sqrt(jnp.sum(pe * pe, axis=1, keepdims=True))
    pe_ref[...] = pe / jnp.maximum(n, 1e-12)
    out_s_ref[...] = jnp.full((bq, _TOPK), _NEG, jnp.float32)
    out_i_ref[...] = jnp.zeros((bq, _TOPK), jnp.int32)

  c = corpus_ref[...]
  n = jnp.sqrt(jnp.sum(c * c, axis=1, keepdims=True))
  cn = c / jnp.maximum(n, 1e-12)
  s = lax.dot_general(pe_ref[...], cn, (((1,), (1,)), ((), ())),
                      preferred_element_type=jnp.float32)
  col = lax.broadcasted_iota(jnp.int32, (bq, dt), 1)
  s = jnp.where(col + d * dt < ndocs, s, _NEG)

  tile_s, tile_i = _extract_top8(s, col, d * dt)

  cs = jnp.concatenate([out_s_ref[...], tile_s], axis=1)
  ci = jnp.concatenate([out_i_ref[...], tile_i], axis=1)
  pos = lax.broadcasted_iota(jnp.int32, (bq, 2 * _TOPK), 1)
  ns, ni = [], []
  for _ in range(_TOPK):
    m = jnp.max(cs, axis=1, keepdims=True)
    eq = cs == m
    p = jnp.min(jnp.where(eq, pos, _BIGI), axis=1, keepdims=True)
    ns.append(m)
    ni.append(jnp.sum(jnp.where(pos == p, ci, 0), axis=1, keepdims=True))
    cs = jnp.where(pos == p, _NEG, cs)
  out_s_ref[...] = jnp.concatenate(ns, axis=1)
  out_i_ref[...] = jnp.concatenate(ni, axis=1)


def _retrieval_topk(patient, wpe_t, bpe, corpus):
  b, _ = patient.shape
  ndocs, emb = corpus.shape
  dt = 512
  qt = min(512, b)
  ndocs_pad = (ndocs + dt - 1) // dt * dt
  corpus_pad = jnp.pad(corpus, ((0, ndocs_pad - ndocs), (0, 0)))
  grid = (b // qt, ndocs_pad // dt)

  return pl.pallas_call(
      functools.partial(_stage1_body, ndocs, dt),
      grid=grid,
      in_specs=[
          pl.BlockSpec((qt, patient.shape[1]), lambda q, d: (q, 0)),
          pl.BlockSpec(wpe_t.shape, lambda q, d: (0, 0)),
          pl.BlockSpec(bpe.shape, lambda q, d: (0, 0)),
          pl.BlockSpec((dt, emb), lambda q, d: (d, 0)),
      ],
      out_specs=[
          pl.BlockSpec((qt, _TOPK), lambda q, d: (q, 0)),
          pl.BlockSpec((qt, _TOPK), lambda q, d: (q, 0)),
      ],
      out_shape=[
          jax.ShapeDtypeStruct((b, _TOPK), jnp.float32),
          jax.ShapeDtypeStruct((b, _TOPK), jnp.int32),
      ],
      scratch_shapes=[pltpu.VMEM((qt, emb), jnp.float32)],
      compiler_params=pltpu.CompilerParams(
          dimension_semantics=("arbitrary", "arbitrary")),
  )(patient, wpe_t, bpe, corpus_pad)


def _sc_gather(table, idx):
  nrows, emb = table.shape
  total = idx.shape[0]
  nw = 32
  b_per_w = total // nw
  ch = 128 if b_per_w % 128 == 0 else b_per_w
  nchunk = b_per_w // ch
  mesh = plsc.VectorSubcoreMesh(core_axis_name="c", subcore_axis_name="s")

  @functools.partial(
      pl.kernel,
      out_type=jax.ShapeDtypeStruct((total, emb), jnp.float32),
      mesh=mesh,
      scratch_types=[
          pltpu.VMEM((ch,), jnp.int32),
          pltpu.VMEM((ch, emb), jnp.float32),
          pltpu.SemaphoreType.DMA,
      ],
  )
  def gather_kernel(table_hbm, idx_hbm, out_hbm, idx_v, rows_v, sem):
    wid = lax.axis_index("s") * 2 + lax.axis_index("c")

    def body(j, carry):
      base = wid * b_per_w + j * ch
      pltpu.sync_copy(idx_hbm.at[pl.ds(base, ch)], idx_v)
      pltpu.async_copy(table_hbm.at[idx_v], rows_v, sem).wait()
      pltpu.sync_copy(rows_v, out_hbm.at[pl.ds(base, ch)])
      return carry

    lax.fori_loop(0, nchunk, body, 0)

  return gather_kernel(table, idx)


def _mlp_body(instr_ref, flat_ref, conf_ref,
              w1a_ref, w1b_ref, b1_ref, w2_ref, b2_ref,
              wsa_ref, wsb_ref, wsc_ref, bs1_ref,
              ws2_ref, bs2_ref, ws3_ref, bs3_ref,
              wiv_ref, biv_ref,
              out_ref, pt_ref, iv_ref):
  instr = instr_ref[...]
  flat = flat_ref[...]
  mm = lambda a, b: jnp.dot(a, b, preferred_element_type=jnp.float32)
  h = jnp.maximum(mm(instr, w1a_ref[...]) + mm(flat, w1b_ref[...])
                  + b1_ref[...], 0.0)
  pt = mm(h, w2_ref[...]) + b2_ref[...]
  h2 = jnp.maximum(mm(pt, wsa_ref[...]) + mm(conf_ref[...], wsb_ref[...])
                   + mm(flat, wsc_ref[...]) + bs1_ref[...], 0.0)
  h3 = jnp.maximum(mm(h2, ws2_ref[...]) + bs2_ref[...], 0.0)
  out_ref[...] = mm(h3, ws3_ref[...]) + bs3_ref[...]
  pt_ref[...] = pt
  iv_ref[...] = mm(instr, wiv_ref[...]) + biv_ref[...]


def _mlp(instruments, flat, confounders, weights):
  b = flat.shape[0]
  bt = min(512, b)
  (w1a, w1b, b1, w2, b2, wsa, wsb, wsc, bs1, ws2, bs2, ws3, bs3,
   wiv, biv) = weights
  fixed = lambda a: pl.BlockSpec(a.shape, lambda i: (0,) * a.ndim)
  row = lambda a: pl.BlockSpec((bt, a.shape[1]), lambda i: (i, 0))

  return pl.pallas_call(
      _mlp_body,
      grid=(b // bt,),
      in_specs=[row(instruments), row(flat), row(confounders)]
      + [fixed(w) for w in weights],
      out_specs=[
          pl.BlockSpec((bt, 1), lambda i: (i, 0)),
          pl.BlockSpec((bt, 2), lambda i: (i, 0)),
          pl.BlockSpec((bt, 2), lambda i: (i, 0)),
      ],
      out_shape=[
          jax.ShapeDtypeStruct((b, 1), jnp.float32),
          jax.ShapeDtypeStruct((b, 2), jnp.float32),
          jax.ShapeDtypeStruct((b, 2), jnp.float32),
      ],
  )(instruments, flat, confounders, *weights)


def kernel(patient, treatment, confounders, corpus_embeddings,
           W_pe, b_pe, W_fs1, b_fs1, W_fs2, b_fs2,
           W_ss1, b_ss1, W_ss2, b_ss2, W_ss3, b_ss3,
           W_iv, b_iv, instruments):
  b = patient.shape[0]
  emb = corpus_embeddings.shape[1]
  k = _TOPK
  instr_d = instruments.shape[1]
  conf_d = confounders.shape[1]

  scores8, idx8 = _retrieval_topk(
      patient, W_pe.T, b_pe.reshape(1, -1), corpus_embeddings)

  flat = _sc_gather(corpus_embeddings, idx8.reshape(-1)).reshape(b, k * emb)

  weights = (
      W_fs1[:, :instr_d].T, W_fs1[:, instr_d:].T, b_fs1.reshape(1, -1),
      W_fs2.T, b_fs2.reshape(1, -1),
      W_ss1[:, :2].T, W_ss1[:, 2:2 + conf_d].T, W_ss1[:, 2 + conf_d:].T,
      b_ss1.reshape(1, -1),
      W_ss2.T, b_ss2.reshape(1, -1), W_ss3.T, b_ss3.reshape(1, -1),
      W_iv.T, b_iv.reshape(1, -1),
  )
  outcome, pt, iv = _mlp(instruments, flat, confounders, weights)

  return (outcome, scores8, idx8, pt, instruments, iv)


# speedup vs baseline: 4.1840x; 4.1840x over previous
import functools

import jax
import jax.numpy as jnp
from jax import lax
from jax.experimental import pallas as pl
from jax.experimental.pallas import tpu as pltpu
from jax.experimental.pallas import tpu_sc as plsc

_TOPK = 8
_NEG = -1e30
_BIGF = 1e9
_DT = 512
_CHUNK = 256
_CPT = _DT // _CHUNK


def _stage1_body(patient_ref, wpet_ref, bpe_ref, corpus_ref,
                 scores_ref, cm_ref, pe_ref):
  d = pl.program_id(1)
  bq = patient_ref.shape[0]
  w = cm_ref.shape[1]

  @pl.when(d == 0)
  def _init():
    pe = jnp.dot(patient_ref[...], wpet_ref[...],
                 preferred_element_type=jnp.float32) + bpe_ref[...]
    n = jnp.sqrt(jnp.sum(pe * pe, axis=1, keepdims=True))
    pe_ref[...] = pe / jnp.maximum(n, 1e-12)
    cm_ref[...] = jnp.full((bq, w), _NEG, jnp.float32)

  c = corpus_ref[...]
  n = jnp.sqrt(jnp.sum(c * c, axis=1, keepdims=True))
  cn = c / jnp.maximum(n, 1e-12)
  s = lax.dot_general(pe_ref[...], cn, (((1,), (1,)), ((), ())),
                      preferred_element_type=jnp.float32)
  scores_ref[...] = s

  lane = lax.broadcasted_iota(jnp.int32, (bq, w), 1)
  cm = cm_ref[...]
  for j in range(_CPT):
    cj = jnp.max(s[:, j * _CHUNK:(j + 1) * _CHUNK], axis=1, keepdims=True)
    cm = jnp.where(lane == d * _CPT + j, cj, cm)
  cm_ref[...] = cm


def _similarity_spill(patient, wpe_t, bpe, corpus):
  b = patient.shape[0]
  ndocs, emb = corpus.shape
  qt = min(512, b)
  ndocs_pad = (ndocs + _DT - 1) // _DT * _DT
  corpus_pad = jnp.pad(corpus, ((0, ndocs_pad - ndocs), (0, 0)))
  nchunk = ndocs_pad // _CHUNK
  w = (nchunk + 127) // 128 * 128
  grid = (b // qt, ndocs_pad // _DT)

  scores, cm = pl.pallas_call(
      _stage1_body,
      grid=grid,
      in_specs=[
          pl.BlockSpec((qt, patient.shape[1]), lambda q, d: (q, 0)),
          pl.BlockSpec(wpe_t.shape, lambda q, d: (0, 0)),
          pl.BlockSpec(bpe.shape, lambda q, d: (0, 0)),
          pl.BlockSpec((_DT, emb), lambda q, d: (d, 0)),
      ],
      out_specs=[
          pl.BlockSpec((qt, _DT), lambda q, d: (q, d)),
          pl.BlockSpec((qt, w), lambda q, d: (q, 0)),
      ],
      out_shape=[
          jax.ShapeDtypeStruct((b, ndocs_pad), jnp.float32),
          jax.ShapeDtypeStruct((b, w), jnp.float32),
      ],
      scratch_shapes=[pltpu.VMEM((qt, emb), jnp.float32)],
      compiler_params=pltpu.CompilerParams(
          dimension_semantics=("arbitrary", "arbitrary")),
  )(patient, wpe_t, bpe, corpus_pad)
  return scores, cm, nchunk


def _phaseb_body(ndocs, nchunk, cm_ref, rows_ref, base_ref):
  q = pl.program_id(0)
  bq, w = cm_ref.shape
  fp = -(-ndocs // _CHUNK)
  lane = lax.broadcasted_iota(jnp.int32, (bq, w), 1).astype(jnp.float32)
  cm = jnp.where(lane < fp, cm_ref[...], _NEG)
  rowi = (lax.broadcasted_iota(jnp.int32, (bq, _TOPK), 0).astype(jnp.float32)
          + q * bq)

  cids, vals = [], []
  for _ in range(_TOPK):
    m = jnp.max(cm, axis=1, keepdims=True)
    eq = cm == m
    cid = jnp.min(jnp.where(eq, lane, _BIGF), axis=1, keepdims=True)
    cids.append(cid)
    cm = jnp.where(lane == cid, _NEG, cm)
  cid8 = jnp.concatenate(cids, axis=1)
  rows_ref[...] = (rowi * nchunk + cid8).astype(jnp.int32)
  base_ref[...] = (cid8 * _CHUNK).astype(jnp.int32)


def _top_chunks(cm, ndocs, nchunk):
  b, w = cm.shape
  qt = min(512, b)
  return pl.pallas_call(
      functools.partial(_phaseb_body, ndocs, nchunk),
      grid=(b // qt,),
      in_specs=[pl.BlockSpec((qt, w), lambda q: (q, 0))],
      out_specs=[
          pl.BlockSpec((qt, _TOPK), lambda q: (q, 0)),
          pl.BlockSpec((qt, _TOPK), lambda q: (q, 0)),
      ],
      out_shape=[
          jax.ShapeDtypeStruct((b, _TOPK), jnp.int32),
          jax.ShapeDtypeStruct((b, _TOPK), jnp.int32),
      ],
  )(cm)


def _phased_body(ndocs, g_ref, base_ref, s8_ref, i8_ref):
  bq = g_ref.shape[0]
  kw = _TOPK * _CHUNK
  g = g_ref[...]
  basef = base_ref[...].astype(jnp.float32)
  offs = jnp.astype(
      lax.broadcasted_iota(jnp.int32, (bq, kw), 1) & (_CHUNK - 1),
      jnp.float32)
  baseexp = jnp.concatenate(
      [jnp.broadcast_to(basef[:, j:j + 1], (bq, _CHUNK))
       for j in range(_TOPK)], axis=1)
  gidx = baseexp + offs
  g = jnp.where(gidx < ndocs, g, _NEG)

  ss, ii = [], []
  for _ in range(_TOPK):
    m = jnp.max(g, axis=1, keepdims=True)
    eq = g == m
    gi = jnp.min(jnp.where(eq, gidx, _BIGF), axis=1, keepdims=True)
    ss.append(m)
    ii.append(gi)
    g = jnp.where(gidx == gi, _NEG, g)
  s8_ref[...] = jnp.concatenate(ss, axis=1)
  i8_ref[...] = jnp.concatenate(ii, axis=1).astype(jnp.int32)


def _top_docs(gathered, base, ndocs):
  b = gathered.shape[0]
  qt = min(512, b)
  kw = _TOPK * _CHUNK
  return pl.pallas_call(
      functools.partial(_phased_body, ndocs),
      grid=(b // qt,),
      in_specs=[
          pl.BlockSpec((qt, kw), lambda q: (q, 0)),
          pl.BlockSpec((qt, _TOPK), lambda q: (q, 0)),
      ],
      out_specs=[
          pl.BlockSpec((qt, _TOPK), lambda q: (q, 0)),
          pl.BlockSpec((qt, _TOPK), lambda q: (q, 0)),
      ],
      out_shape=[
          jax.ShapeDtypeStruct((b, _TOPK), jnp.float32),
          jax.ShapeDtypeStruct((b, _TOPK), jnp.int32),
      ],
  )(gathered, base)


def _sc_gather(table, idx):
  nrows, width = table.shape
  total = idx.shape[0]
  nw = 32
  b_per_w = total // nw
  ch = 128 if b_per_w % 128 == 0 else b_per_w
  nchunk = b_per_w // ch
  mesh = plsc.VectorSubcoreMesh(core_axis_name="c", subcore_axis_name="s")

  @functools.partial(
      pl.kernel,
      out_type=jax.ShapeDtypeStruct((total, width), jnp.float32),
      mesh=mesh,
      scratch_types=[
          pltpu.VMEM((ch,), jnp.int32),
          pltpu.VMEM((ch, width), jnp.float32),
          pltpu.SemaphoreType.DMA,
      ],
  )
  def gather_kernel(table_hbm, idx_hbm, out_hbm, idx_v, rows_v, sem):
    wid = lax.axis_index("s") * 2 + lax.axis_index("c")

    def body(j, carry):
      base = wid * b_per_w + j * ch
      pltpu.sync_copy(idx_hbm.at[pl.ds(base, ch)], idx_v)
      pltpu.async_copy(table_hbm.at[idx_v], rows_v, sem).wait()
      pltpu.sync_copy(rows_v, out_hbm.at[pl.ds(base, ch)])
      return carry

    lax.fori_loop(0, nchunk, body, 0)

  return gather_kernel(table, idx)


def _mlp_body(instr_ref, flat_ref, conf_ref,
              w1a_ref, w1b_ref, b1_ref, w2_ref, b2_ref,
              wsa_ref, wsb_ref, wsc_ref, bs1_ref,
              ws2_ref, bs2_ref, ws3_ref, bs3_ref,
              wiv_ref, biv_ref,
              out_ref, pt_ref, iv_ref):
  instr = instr_ref[...]
  flat = flat_ref[...]
  mm = lambda a, b: jnp.dot(a, b, preferred_element_type=jnp.float32)
  h = jnp.maximum(mm(instr, w1a_ref[...]) + mm(flat, w1b_ref[...])
                  + b1_ref[...], 0.0)
  pt = mm(h, w2_ref[...]) + b2_ref[...]
  h2 = jnp.maximum(mm(pt, wsa_ref[...]) + mm(conf_ref[...], wsb_ref[...])
                   + mm(flat, wsc_ref[...]) + bs1_ref[...], 0.0)
  h3 = jnp.maximum(mm(h2, ws2_ref[...]) + bs2_ref[...], 0.0)
  out_ref[...] = mm(h3, ws3_ref[...]) + bs3_ref[...]
  pt_ref[...] = pt
  iv_ref[...] = mm(instr, wiv_ref[...]) + biv_ref[...]


def _mlp(instruments, flat, confounders, weights):
  b = flat.shape[0]
  bt = min(512, b)
  fixed = lambda a: pl.BlockSpec(a.shape, lambda i: (0,) * a.ndim)
  row = lambda a: pl.BlockSpec((bt, a.shape[1]), lambda i: (i, 0))

  return pl.pallas_call(
      _mlp_body,
      grid=(b // bt,),
      in_specs=[row(instruments), row(flat), row(confounders)]
      + [fixed(w) for w in weights],
      out_specs=[
          pl.BlockSpec((bt, 1), lambda i: (i, 0)),
          pl.BlockSpec((bt, 2), lambda i: (i, 0)),
          pl.BlockSpec((bt, 2), lambda i: (i, 0)),
      ],
      out_shape=[
          jax.ShapeDtypeStruct((b, 1), jnp.float32),
          jax.ShapeDtypeStruct((b, 2), jnp.float32),
          jax.ShapeDtypeStruct((b, 2), jnp.float32),
      ],
  )(instruments, flat, confounders, *weights)


def kernel(patient, treatment, confounders, corpus_embeddings,
           W_pe, b_pe, W_fs1, b_fs1, W_fs2, b_fs2,
           W_ss1, b_ss1, W_ss2, b_ss2, W_ss3, b_ss3,
           W_iv, b_iv, instruments):
  b = patient.shape[0]
  ndocs, emb = corpus_embeddings.shape
  k = _TOPK
  instr_d = instruments.shape[1]
  conf_d = confounders.shape[1]

  scores, cm, nchunk = _similarity_spill(
      patient, W_pe.T, b_pe.reshape(1, -1), corpus_embeddings)
  rows, base = _top_chunks(cm, ndocs, nchunk)
  cand = _sc_gather(scores.reshape(b * nchunk, _CHUNK), rows.reshape(-1))
  scores8, idx8 = _top_docs(cand.reshape(b, k * _CHUNK), base, ndocs)

  flat = _sc_gather(corpus_embeddings, idx8.reshape(-1)).reshape(b, k * emb)

  weights = (
      W_fs1[:, :instr_d].T, W_fs1[:, instr_d:].T, b_fs1.reshape(1, -1),
      W_fs2.T, b_fs2.reshape(1, -1),
      W_ss1[:, :2].T, W_ss1[:, 2:2 + conf_d].T, W_ss1[:, 2 + conf_d:].T,
      b_ss1.reshape(1, -1),
      W_ss2.T, b_ss2.reshape(1, -1), W_ss3.T, b_ss3.reshape(1, -1),
      W_iv.T, b_iv.reshape(1, -1),
  )
  outcome, pt, iv = _mlp(instruments, flat, confounders, weights)

  return (outcome, scores8, idx8, pt, instruments, iv)


# speedup vs baseline: 6.6399x; 1.5869x over previous
import functools

import jax
import jax.numpy as jnp
from jax import lax
from jax.experimental import pallas as pl
from jax.experimental.pallas import tpu as pltpu
from jax.experimental.pallas import tpu_sc as plsc

_TOPK = 8
_NEG = -1e30
_BIGF = 1e9
_DT = 512
_CHUNK = 256
_CPT = _DT // _CHUNK


def _stage1_body(patient_ref, wpet_ref, bpe_ref, corpus_ref,
                 scores_ref, cm_ref, pe_ref):
  d = pl.program_id(1)
  bq = patient_ref.shape[0]
  w = cm_ref.shape[1]

  @pl.when(d == 0)
  def _init():
    pe = jnp.dot(patient_ref[...], wpet_ref[...],
                 preferred_element_type=jnp.float32) + bpe_ref[...]
    n = jnp.sqrt(jnp.sum(pe * pe, axis=1, keepdims=True))
    pe_ref[...] = pe / jnp.maximum(n, 1e-12)
    cm_ref[...] = jnp.full((bq, w), _NEG, jnp.float32)

  c = corpus_ref[...]
  n = jnp.sqrt(jnp.sum(c * c, axis=1, keepdims=True))
  cn = c / jnp.maximum(n, 1e-12)
  s = lax.dot_general(pe_ref[...], cn, (((1,), (1,)), ((), ())),
                      preferred_element_type=jnp.float32)
  for j in range(_CPT):
    scores_ref[j] = s[:, j * _CHUNK:(j + 1) * _CHUNK]

  lane = lax.broadcasted_iota(jnp.int32, (bq, w), 1)
  cm = cm_ref[...]
  for j in range(_CPT):
    cj = jnp.max(s[:, j * _CHUNK:(j + 1) * _CHUNK], axis=1, keepdims=True)
    cm = jnp.where(lane == d * _CPT + j, cj, cm)
  cm_ref[...] = cm


def _similarity_spill(patient, wpe_t, bpe, corpus):
  b = patient.shape[0]
  ndocs, emb = corpus.shape
  qt = min(512, b)
  ndocs_pad = (ndocs + _DT - 1) // _DT * _DT
  corpus_pad = jnp.pad(corpus, ((0, ndocs_pad - ndocs), (0, 0)))
  nchunk = ndocs_pad // _CHUNK
  w = (nchunk + 127) // 128 * 128
  grid = (b // qt, ndocs_pad // _DT)

  scores, cm = pl.pallas_call(
      _stage1_body,
      grid=grid,
      in_specs=[
          pl.BlockSpec((qt, patient.shape[1]), lambda q, d: (q, 0)),
          pl.BlockSpec(wpe_t.shape, lambda q, d: (0, 0)),
          pl.BlockSpec(bpe.shape, lambda q, d: (0, 0)),
          pl.BlockSpec((_DT, emb), lambda q, d: (d, 0)),
      ],
      out_specs=[
          pl.BlockSpec((_CPT, qt, _CHUNK), lambda q, d: (d, q, 0)),
          pl.BlockSpec((qt, w), lambda q, d: (q, 0)),
      ],
      out_shape=[
          jax.ShapeDtypeStruct((nchunk, b, _CHUNK), jnp.float32),
          jax.ShapeDtypeStruct((b, w), jnp.float32),
      ],
      scratch_shapes=[pltpu.VMEM((qt, emb), jnp.float32)],
      compiler_params=pltpu.CompilerParams(
          dimension_semantics=("arbitrary", "arbitrary")),
  )(patient, wpe_t, bpe, corpus_pad)
  return scores, cm, nchunk


def _phaseb_body(ndocs, nchunk, cm_ref, rows_ref, base_ref):
  q = pl.program_id(0)
  bq, w = cm_ref.shape
  fp = -(-ndocs // _CHUNK)
  lane = lax.broadcasted_iota(jnp.int32, (bq, w), 1).astype(jnp.float32)
  cm = jnp.where(lane < fp, cm_ref[...], _NEG)
  rowi = (lax.broadcasted_iota(jnp.int32, (bq, _TOPK), 0).astype(jnp.float32)
          + q * bq)

  cids, vals = [], []
  for _ in range(_TOPK):
    m = jnp.max(cm, axis=1, keepdims=True)
    eq = cm == m
    cid = jnp.min(jnp.where(eq, lane, _BIGF), axis=1, keepdims=True)
    cids.append(cid)
    cm = jnp.where(lane == cid, _NEG, cm)
  cid8 = jnp.concatenate(cids, axis=1)
  rows_ref[...] = (cid8 * (bq * pl.num_programs(0)) + rowi).astype(jnp.int32)
  base_ref[...] = (cid8 * _CHUNK).astype(jnp.int32)


def _top_chunks(cm, ndocs, nchunk):
  b, w = cm.shape
  qt = min(512, b)
  return pl.pallas_call(
      functools.partial(_phaseb_body, ndocs, nchunk),
      grid=(b // qt,),
      in_specs=[pl.BlockSpec((qt, w), lambda q: (q, 0))],
      out_specs=[
          pl.BlockSpec((qt, _TOPK), lambda q: (q, 0)),
          pl.BlockSpec((qt, _TOPK), lambda q: (q, 0)),
      ],
      out_shape=[
          jax.ShapeDtypeStruct((b, _TOPK), jnp.int32),
          jax.ShapeDtypeStruct((b, _TOPK), jnp.int32),
      ],
  )(cm)


def _phased_body(ndocs, g_ref, base_ref, s8_ref, i8_ref):
  bq = g_ref.shape[0]
  kw = _TOPK * _CHUNK
  g = g_ref[...]
  basef = base_ref[...].astype(jnp.float32)
  offs = jnp.astype(
      lax.broadcasted_iota(jnp.int32, (bq, kw), 1) & (_CHUNK - 1),
      jnp.float32)
  baseexp = jnp.concatenate(
      [jnp.broadcast_to(basef[:, j:j + 1], (bq, _CHUNK))
       for j in range(_TOPK)], axis=1)
  gidx = baseexp + offs
  g = jnp.where(gidx < ndocs, g, _NEG)

  ss, ii = [], []
  for _ in range(_TOPK):
    m = jnp.max(g, axis=1, keepdims=True)
    eq = g == m
    gi = jnp.min(jnp.where(eq, gidx, _BIGF), axis=1, keepdims=True)
    ss.append(m)
    ii.append(gi)
    g = jnp.where(gidx == gi, _NEG, g)
  s8_ref[...] = jnp.concatenate(ss, axis=1)
  i8_ref[...] = jnp.concatenate(ii, axis=1).astype(jnp.int32)


def _top_docs(gathered, base, ndocs):
  b = gathered.shape[0]
  qt = min(512, b)
  kw = _TOPK * _CHUNK
  return pl.pallas_call(
      functools.partial(_phased_body, ndocs),
      grid=(b // qt,),
      in_specs=[
          pl.BlockSpec((qt, kw), lambda q: (q, 0)),
          pl.BlockSpec((qt, _TOPK), lambda q: (q, 0)),
      ],
      out_specs=[
          pl.BlockSpec((qt, _TOPK), lambda q: (q, 0)),
          pl.BlockSpec((qt, _TOPK), lambda q: (q, 0)),
      ],
      out_shape=[
          jax.ShapeDtypeStruct((b, _TOPK), jnp.float32),
          jax.ShapeDtypeStruct((b, _TOPK), jnp.int32),
      ],
  )(gathered, base)


def _sc_gather(table, idx):
  nrows, width = table.shape
  total = idx.shape[0]
  nw = 32
  b_per_w = total // nw
  ch = 128 if b_per_w % 128 == 0 else b_per_w
  nchunk = b_per_w // ch
  mesh = plsc.VectorSubcoreMesh(core_axis_name="c", subcore_axis_name="s")

  @functools.partial(
      pl.kernel,
      out_type=jax.ShapeDtypeStruct((total, width), jnp.float32),
      mesh=mesh,
      scratch_types=[
          pltpu.VMEM((ch,), jnp.int32),
          pltpu.VMEM((ch, width), jnp.float32),
          pltpu.SemaphoreType.DMA,
      ],
  )
  def gather_kernel(table_hbm, idx_hbm, out_hbm, idx_v, rows_v, sem):
    wid = lax.axis_index("s") * 2 + lax.axis_index("c")

    def body(j, carry):
      base = wid * b_per_w + j * ch
      pltpu.sync_copy(idx_hbm.at[pl.ds(base, ch)], idx_v)
      pltpu.async_copy(table_hbm.at[idx_v], rows_v, sem).wait()
      pltpu.sync_copy(rows_v, out_hbm.at[pl.ds(base, ch)])
      return carry

    lax.fori_loop(0, nchunk, body, 0)

  return gather_kernel(table, idx)


def _mlp_body(instr_ref, flat_ref, conf_ref,
              w1a_ref, w1b_ref, b1_ref, w2_ref, b2_ref,
              wsa_ref, wsb_ref, wsc_ref, bs1_ref,
              ws2_ref, bs2_ref, ws3_ref, bs3_ref,
              wiv_ref, biv_ref,
              out_ref, pt_ref, iv_ref):
  instr = instr_ref[...]
  flat = flat_ref[...]
  mm = lambda a, b: jnp.dot(a, b, preferred_element_type=jnp.float32)
  h = jnp.maximum(mm(instr, w1a_ref[...]) + mm(flat, w1b_ref[...])
                  + b1_ref[...], 0.0)
  pt = mm(h, w2_ref[...]) + b2_ref[...]
  h2 = jnp.maximum(mm(pt, wsa_ref[...]) + mm(conf_ref[...], wsb_ref[...])
                   + mm(flat, wsc_ref[...]) + bs1_ref[...], 0.0)
  h3 = jnp.maximum(mm(h2, ws2_ref[...]) + bs2_ref[...], 0.0)
  out_ref[...] = mm(h3, ws3_ref[...]) + bs3_ref[...]
  pt_ref[...] = pt
  iv_ref[...] = mm(instr, wiv_ref[...]) + biv_ref[...]


def _mlp(instruments, flat, confounders, weights):
  b = flat.shape[0]
  bt = min(512, b)
  fixed = lambda a: pl.BlockSpec(a.shape, lambda i: (0,) * a.ndim)
  row = lambda a: pl.BlockSpec((bt, a.shape[1]), lambda i: (i, 0))

  return pl.pallas_call(
      _mlp_body,
      grid=(b // bt,),
      in_specs=[row(instruments), row(flat), row(confounders)]
      + [fixed(w) for w in weights],
      out_specs=[
          pl.BlockSpec((bt, 1), lambda i: (i, 0)),
          pl.BlockSpec((bt, 2), lambda i: (i, 0)),
          pl.BlockSpec((bt, 2), lambda i: (i, 0)),
      ],
      out_shape=[
          jax.ShapeDtypeStruct((b, 1), jnp.float32),
          jax.ShapeDtypeStruct((b, 2), jnp.float32),
          jax.ShapeDtypeStruct((b, 2), jnp.float32),
      ],
  )(instruments, flat, confounders, *weights)


def kernel(patient, treatment, confounders, corpus_embeddings,
           W_pe, b_pe, W_fs1, b_fs1, W_fs2, b_fs2,
           W_ss1, b_ss1, W_ss2, b_ss2, W_ss3, b_ss3,
           W_iv, b_iv, instruments):
  b = patient.shape[0]
  ndocs, emb = corpus_embeddings.shape
  k = _TOPK
  instr_d = instruments.shape[1]
  conf_d = confounders.shape[1]

  scores, cm, nchunk = _similarity_spill(
      patient, W_pe.T, b_pe.reshape(1, -1), corpus_embeddings)
  rows, base = _top_chunks(cm, ndocs, nchunk)
  cand = _sc_gather(scores.reshape(nchunk * b, _CHUNK), rows.reshape(-1))
  scores8, idx8 = _top_docs(cand.reshape(b, k * _CHUNK), base, ndocs)

  flat = _sc_gather(corpus_embeddings, idx8.reshape(-1)).reshape(b, k * emb)

  weights = (
      W_fs1[:, :instr_d].T, W_fs1[:, instr_d:].T, b_fs1.reshape(1, -1),
      W_fs2.T, b_fs2.reshape(1, -1),
      W_ss1[:, :2].T, W_ss1[:, 2:2 + conf_d].T, W_ss1[:, 2 + conf_d:].T,
      b_ss1.reshape(1, -1),
      W_ss2.T, b_ss2.reshape(1, -1), W_ss3.T, b_ss3.reshape(1, -1),
      W_iv.T, b_iv.reshape(1, -1),
  )
  outcome, pt, iv = _mlp(instruments, flat, confounders, weights)

  return (outcome, scores8, idx8, pt, instruments, iv)


# speedup vs baseline: 6.7279x; 1.0133x over previous
import functools

import jax
import jax.numpy as jnp
from jax import lax
from jax.experimental import pallas as pl
from jax.experimental.pallas import tpu as pltpu
from jax.experimental.pallas import tpu_sc as plsc

_TOPK = 8
_NEG = -1e30
_BIGF = 1e9
_DT = 512
_CHUNK = 256
_CPT = _DT // _CHUNK


def _stage1_body(pe_ref, corpus_ref, scores_ref, cm_ref):
  d = pl.program_id(1)
  bq = pe_ref.shape[0]
  w = cm_ref.shape[1]

  @pl.when(d == 0)
  def _init():
    cm_ref[...] = jnp.full((bq, w), _NEG, jnp.float32)

  s = lax.dot_general(pe_ref[...], corpus_ref[...], (((1,), (1,)), ((), ())),
                      preferred_element_type=jnp.float32)
  for j in range(_CPT):
    scores_ref[j] = s[:, j * _CHUNK:(j + 1) * _CHUNK]

  lane = lax.broadcasted_iota(jnp.int32, (bq, w), 1)
  cm = cm_ref[...]
  for j in range(_CPT):
    cj = jnp.max(s[:, j * _CHUNK:(j + 1) * _CHUNK], axis=1, keepdims=True)
    cm = jnp.where(lane == d * _CPT + j, cj, cm)
  cm_ref[...] = cm


def _similarity_spill(pe_norm, corpus_norm):
  b, emb = pe_norm.shape
  ndocs = corpus_norm.shape[0]
  qt = min(512, b)
  ndocs_pad = (ndocs + _DT - 1) // _DT * _DT
  corpus_pad = jnp.pad(corpus_norm, ((0, ndocs_pad - ndocs), (0, 0)))
  nchunk = ndocs_pad // _CHUNK
  w = (nchunk + 127) // 128 * 128
  grid = (b // qt, ndocs_pad // _DT)

  scores, cm = pl.pallas_call(
      _stage1_body,
      grid=grid,
      in_specs=[
          pl.BlockSpec((qt, emb), lambda q, d: (q, 0)),
          pl.BlockSpec((_DT, emb), lambda q, d: (d, 0)),
      ],
      out_specs=[
          pl.BlockSpec((_CPT, qt, _CHUNK), lambda q, d: (d, q, 0)),
          pl.BlockSpec((qt, w), lambda q, d: (q, 0)),
      ],
      out_shape=[
          jax.ShapeDtypeStruct((nchunk, b, _CHUNK), jnp.float32),
          jax.ShapeDtypeStruct((b, w), jnp.float32),
      ],
      compiler_params=pltpu.CompilerParams(
          dimension_semantics=("arbitrary", "arbitrary")),
  )(pe_norm, corpus_pad)
  return scores, cm, nchunk


def _phaseb_body(ndocs, nchunk, cm_ref, rows_ref, base_ref):
  q = pl.program_id(0)
  bq, w = cm_ref.shape
  fp = -(-ndocs // _CHUNK)
  lane = lax.broadcasted_iota(jnp.int32, (bq, w), 1).astype(jnp.float32)
  cm = jnp.where(lane < fp, cm_ref[...], _NEG)
  rowi = (lax.broadcasted_iota(jnp.int32, (bq, _TOPK), 0).astype(jnp.float32)
          + q * bq)

  cids, vals = [], []
  for _ in range(_TOPK):
    m = jnp.max(cm, axis=1, keepdims=True)
    eq = cm == m
    cid = jnp.min(jnp.where(eq, lane, _BIGF), axis=1, keepdims=True)
    cids.append(cid)
    cm = jnp.where(lane == cid, _NEG, cm)
  cid8 = jnp.concatenate(cids, axis=1)
  rows_ref[...] = (cid8 * (bq * pl.num_programs(0)) + rowi).astype(jnp.int32)
  base_ref[...] = (cid8 * _CHUNK).astype(jnp.int32)


def _top_chunks(cm, ndocs, nchunk):
  b, w = cm.shape
  qt = min(512, b)
  return pl.pallas_call(
      functools.partial(_phaseb_body, ndocs, nchunk),
      grid=(b // qt,),
      in_specs=[pl.BlockSpec((qt, w), lambda q: (q, 0))],
      out_specs=[
          pl.BlockSpec((qt, _TOPK), lambda q: (q, 0)),
          pl.BlockSpec((qt, _TOPK), lambda q: (q, 0)),
      ],
      out_shape=[
          jax.ShapeDtypeStruct((b, _TOPK), jnp.int32),
          jax.ShapeDtypeStruct((b, _TOPK), jnp.int32),
      ],
  )(cm)


def _phased_body(ndocs, g_ref, base_ref, s8_ref, i8_ref):
  bq = g_ref.shape[0]
  kw = _TOPK * _CHUNK
  g = g_ref[...]
  basef = base_ref[...].astype(jnp.float32)
  offs = jnp.astype(
      lax.broadcasted_iota(jnp.int32, (bq, kw), 1) & (_CHUNK - 1),
      jnp.float32)
  baseexp = jnp.concatenate(
      [jnp.broadcast_to(basef[:, j:j + 1], (bq, _CHUNK))
       for j in range(_TOPK)], axis=1)
  gidx = baseexp + offs
  g = jnp.where(gidx < ndocs, g, _NEG)

  ss, ii = [], []
  for _ in range(_TOPK):
    m = jnp.max(g, axis=1, keepdims=True)
    eq = g == m
    gi = jnp.min(jnp.where(eq, gidx, _BIGF), axis=1, keepdims=True)
    ss.append(m)
    ii.append(gi)
    g = jnp.where(gidx == gi, _NEG, g)
  s8_ref[...] = jnp.concatenate(ss, axis=1)
  i8_ref[...] = jnp.concatenate(ii, axis=1).astype(jnp.int32)


def _top_docs(gathered, base, ndocs):
  b = gathered.shape[0]
  qt = min(512, b)
  kw = _TOPK * _CHUNK
  return pl.pallas_call(
      functools.partial(_phased_body, ndocs),
      grid=(b // qt,),
      in_specs=[
          pl.BlockSpec((qt, kw), lambda q: (q, 0)),
          pl.BlockSpec((qt, _TOPK), lambda q: (q, 0)),
      ],
      out_specs=[
          pl.BlockSpec((qt, _TOPK), lambda q: (q, 0)),
          pl.BlockSpec((qt, _TOPK), lambda q: (q, 0)),
      ],
      out_shape=[
          jax.ShapeDtypeStruct((b, _TOPK), jnp.float32),
          jax.ShapeDtypeStruct((b, _TOPK), jnp.int32),
      ],
  )(gathered, base)


def _sc_gather(table, idx):
  nrows, width = table.shape
  total = idx.shape[0]
  nw = 32
  b_per_w = total // nw
  ch = 128 if b_per_w % 128 == 0 else b_per_w
  nchunk = b_per_w // ch
  mesh = plsc.VectorSubcoreMesh(core_axis_name="c", subcore_axis_name="s")

  @functools.partial(
      pl.kernel,
      out_type=jax.ShapeDtypeStruct((total, width), jnp.float32),
      mesh=mesh,
      scratch_types=[
          pltpu.VMEM((ch,), jnp.int32),
          pltpu.VMEM((ch, width), jnp.float32),
          pltpu.SemaphoreType.DMA,
      ],
  )
  def gather_kernel(table_hbm, idx_hbm, out_hbm, idx_v, rows_v, sem):
    wid = lax.axis_index("s") * 2 + lax.axis_index("c")

    def body(j, carry):
      base = wid * b_per_w + j * ch
      pltpu.sync_copy(idx_hbm.at[pl.ds(base, ch)], idx_v)
      pltpu.async_copy(table_hbm.at[idx_v], rows_v, sem).wait()
      pltpu.sync_copy(rows_v, out_hbm.at[pl.ds(base, ch)])
      return carry

    lax.fori_loop(0, nchunk, body, 0)

  return gather_kernel(table, idx)


def _mlp_body(instr_ref, flat_ref, conf_ref,
              w1a_ref, w1b_ref, b1_ref, w2_ref, b2_ref,
              wsa_ref, wsb_ref, wsc_ref, bs1_ref,
              ws2_ref, bs2_ref, ws3_ref, bs3_ref,
              wiv_ref, biv_ref,
              out_ref, pt_ref, iv_ref):
  instr = instr_ref[...]
  flat = flat_ref[...]
  mm = lambda a, b: jnp.dot(a, b, preferred_element_type=jnp.float32)
  h = jnp.maximum(mm(instr, w1a_ref[...]) + mm(flat, w1b_ref[...])
                  + b1_ref[...], 0.0)
  pt = mm(h, w2_ref[...]) + b2_ref[...]
  h2 = jnp.maximum(mm(pt, wsa_ref[...]) + mm(conf_ref[...], wsb_ref[...])
                   + mm(flat, wsc_ref[...]) + bs1_ref[...], 0.0)
  h3 = jnp.maximum(mm(h2, ws2_ref[...]) + bs2_ref[...], 0.0)
  out_ref[...] = mm(h3, ws3_ref[...]) + bs3_ref[...]
  pt_ref[...] = pt
  iv_ref[...] = mm(instr, wiv_ref[...]) + biv_ref[...]


def _mlp(instruments, flat, confounders, weights):
  b = flat.shape[0]
  bt = min(512, b)
  fixed = lambda a: pl.BlockSpec(a.shape, lambda i: (0,) * a.ndim)
  row = lambda a: pl.BlockSpec((bt, a.shape[1]), lambda i: (i, 0))

  return pl.pallas_call(
      _mlp_body,
      grid=(b // bt,),
      in_specs=[row(instruments), row(flat), row(confounders)]
      + [fixed(w) for w in weights],
      out_specs=[
          pl.BlockSpec((bt, 1), lambda i: (i, 0)),
          pl.BlockSpec((bt, 2), lambda i: (i, 0)),
          pl.BlockSpec((bt, 2), lambda i: (i, 0)),
      ],
      out_shape=[
          jax.ShapeDtypeStruct((b, 1), jnp.float32),
          jax.ShapeDtypeStruct((b, 2), jnp.float32),
          jax.ShapeDtypeStruct((b, 2), jnp.float32),
      ],
  )(instruments, flat, confounders, *weights)


def kernel(patient, treatment, confounders, corpus_embeddings,
           W_pe, b_pe, W_fs1, b_fs1, W_fs2, b_fs2,
           W_ss1, b_ss1, W_ss2, b_ss2, W_ss3, b_ss3,
           W_iv, b_iv, instruments):
  b = patient.shape[0]
  ndocs, emb = corpus_embeddings.shape
  k = _TOPK
  instr_d = instruments.shape[1]
  conf_d = confounders.shape[1]

  pe = patient @ W_pe.T + b_pe
  pe = pe / jnp.clip(jnp.sqrt(jnp.sum(pe * pe, axis=1, keepdims=True)), 1e-12)
  cn = corpus_embeddings / jnp.clip(
      jnp.sqrt(jnp.sum(corpus_embeddings * corpus_embeddings, axis=1,
                       keepdims=True)), 1e-12)

  scores, cm, nchunk = _similarity_spill(pe, cn)
  rows, base = _top_chunks(cm, ndocs, nchunk)
  cand = _sc_gather(scores.reshape(nchunk * b, _CHUNK), rows.reshape(-1))
  scores8, idx8 = _top_docs(cand.reshape(b, k * _CHUNK), base, ndocs)

  flat = _sc_gather(corpus_embeddings, idx8.reshape(-1)).reshape(b, k * emb)

  weights = (
      W_fs1[:, :instr_d].T, W_fs1[:, instr_d:].T, b_fs1.reshape(1, -1),
      W_fs2.T, b_fs2.reshape(1, -1),
      W_ss1[:, :2].T, W_ss1[:, 2:2 + conf_d].T, W_ss1[:, 2 + conf_d:].T,
      b_ss1.reshape(1, -1),
      W_ss2.T, b_ss2.reshape(1, -1), W_ss3.T, b_ss3.reshape(1, -1),
      W_iv.T, b_iv.reshape(1, -1),
  )
  outcome, pt, iv = _mlp(instruments, flat, confounders, weights)

  return (outcome, scores8, idx8, pt, instruments, iv)


# speedup vs baseline: 7.8574x; 1.1679x over previous
import functools

import jax
import jax.numpy as jnp
from jax import lax
from jax.experimental import pallas as pl
from jax.experimental.pallas import tpu as pltpu
from jax.experimental.pallas import tpu_sc as plsc

_TOPK = 8
_NEG = -1e30
_BIGF = 1e9
_DT = 512
_CHUNK = 256
_CPT = _DT // _CHUNK


def _stage1_body(qt, pe_ref, corpus_ref, scores_ref, cm_ref):
  d = pl.program_id(0)
  q = pl.program_id(1)
  w = cm_ref.shape[1]
  qs = pl.ds(q * qt, qt)

  s = lax.dot_general(pe_ref[qs, :], corpus_ref[...], (((1,), (1,)), ((), ())),
                      preferred_element_type=jnp.float32)
  for j in range(_CPT):
    scores_ref[j] = s[:, j * _CHUNK:(j + 1) * _CHUNK]

  lane = lax.broadcasted_iota(jnp.int32, (qt, w), 1)
  prev = jnp.where(d == 0, jnp.full((qt, w), _NEG, jnp.float32), cm_ref[qs, :])
  for j in range(_CPT):
    cj = jnp.max(s[:, j * _CHUNK:(j + 1) * _CHUNK], axis=1, keepdims=True)
    prev = jnp.where(lane == d * _CPT + j, cj, prev)
  cm_ref[qs, :] = prev


def _similarity_spill(pe_norm, corpus_pad):
  b, emb = pe_norm.shape
  ndocs_pad = corpus_pad.shape[0]
  qt = min(512, b)
  nchunk = ndocs_pad // _CHUNK
  w = (nchunk + 127) // 128 * 128
  grid = (ndocs_pad // _DT, b // qt)

  scores, cm = pl.pallas_call(
      functools.partial(_stage1_body, qt),
      grid=grid,
      in_specs=[
          pl.BlockSpec((b, emb), lambda d, q: (0, 0)),
          pl.BlockSpec((_DT, emb), lambda d, q: (d, 0)),
      ],
      out_specs=[
          pl.BlockSpec((_CPT, qt, _CHUNK), lambda d, q: (d, q, 0)),
          pl.BlockSpec((b, w), lambda d, q: (0, 0)),
      ],
      out_shape=[
          jax.ShapeDtypeStruct((nchunk, b, _CHUNK), jnp.float32),
          jax.ShapeDtypeStruct((b, w), jnp.float32),
      ],
      compiler_params=pltpu.CompilerParams(
          dimension_semantics=("arbitrary", "arbitrary")),
  )(pe_norm, corpus_pad)
  return scores, cm, nchunk


def _phaseb_body(ndocs, nchunk, cm_ref, rows_ref, base_ref):
  q = pl.program_id(0)
  bq, w = cm_ref.shape
  fp = -(-ndocs // _CHUNK)
  lane = lax.broadcasted_iota(jnp.int32, (bq, w), 1).astype(jnp.float32)
  cm = jnp.where(lane < fp, cm_ref[...], _NEG)
  rowi = (lax.broadcasted_iota(jnp.int32, (bq, _TOPK), 0).astype(jnp.float32)
          + q * bq)

  cids, vals = [], []
  for _ in range(_TOPK):
    m = jnp.max(cm, axis=1, keepdims=True)
    eq = cm == m
    cid = jnp.min(jnp.where(eq, lane, _BIGF), axis=1, keepdims=True)
    cids.append(cid)
    cm = jnp.where(lane == cid, _NEG, cm)
  cid8 = jnp.concatenate(cids, axis=1)
  rows_ref[...] = (cid8 * (bq * pl.num_programs(0)) + rowi).astype(jnp.int32)
  base_ref[...] = (cid8 * _CHUNK).astype(jnp.int32)


def _top_chunks(cm, ndocs, nchunk):
  b, w = cm.shape
  qt = min(512, b)
  return pl.pallas_call(
      functools.partial(_phaseb_body, ndocs, nchunk),
      grid=(b // qt,),
      in_specs=[pl.BlockSpec((qt, w), lambda q: (q, 0))],
      out_specs=[
          pl.BlockSpec((qt, _TOPK), lambda q: (q, 0)),
          pl.BlockSpec((qt, _TOPK), lambda q: (q, 0)),
      ],
      out_shape=[
          jax.ShapeDtypeStruct((b, _TOPK), jnp.int32),
          jax.ShapeDtypeStruct((b, _TOPK), jnp.int32),
      ],
  )(cm)


def _phased_body(ndocs, g_ref, base_ref, s8_ref, i8_ref):
  bq = g_ref.shape[0]
  kw = _TOPK * _CHUNK
  g = g_ref[...]
  basef = base_ref[...].astype(jnp.float32)
  offs = jnp.astype(
      lax.broadcasted_iota(jnp.int32, (bq, kw), 1) & (_CHUNK - 1),
      jnp.float32)
  baseexp = jnp.concatenate(
      [jnp.broadcast_to(basef[:, j:j + 1], (bq, _CHUNK))
       for j in range(_TOPK)], axis=1)
  gidx = baseexp + offs
  g = jnp.where(gidx < ndocs, g, _NEG)

  ss, ii = [], []
  for _ in range(_TOPK):
    m = jnp.max(g, axis=1, keepdims=True)
    eq = g == m
    gi = jnp.min(jnp.where(eq, gidx, _BIGF), axis=1, keepdims=True)
    ss.append(m)
    ii.append(gi)
    g = jnp.where(gidx == gi, _NEG, g)
  s8_ref[...] = jnp.concatenate(ss, axis=1)
  i8_ref[...] = jnp.concatenate(ii, axis=1).astype(jnp.int32)


def _top_docs(gathered, base, ndocs):
  b = gathered.shape[0]
  qt = min(512, b)
  kw = _TOPK * _CHUNK
  return pl.pallas_call(
      functools.partial(_phased_body, ndocs),
      grid=(b // qt,),
      in_specs=[
          pl.BlockSpec((qt, kw), lambda q: (q, 0)),
          pl.BlockSpec((qt, _TOPK), lambda q: (q, 0)),
      ],
      out_specs=[
          pl.BlockSpec((qt, _TOPK), lambda q: (q, 0)),
          pl.BlockSpec((qt, _TOPK), lambda q: (q, 0)),
      ],
      out_shape=[
          jax.ShapeDtypeStruct((b, _TOPK), jnp.float32),
          jax.ShapeDtypeStruct((b, _TOPK), jnp.int32),
      ],
  )(gathered, base)


def _sc_gather(table, idx):
  nrows, width = table.shape
  total = idx.shape[0]
  nw = 32
  b_per_w = total // nw
  ch = 128 if b_per_w % 128 == 0 else b_per_w
  nchunk = b_per_w // ch
  mesh = plsc.VectorSubcoreMesh(core_axis_name="c", subcore_axis_name="s")

  @functools.partial(
      pl.kernel,
      out_type=jax.ShapeDtypeStruct((total, width), jnp.float32),
      mesh=mesh,
      scratch_types=[
          pltpu.VMEM((ch,), jnp.int32),
          pltpu.VMEM((ch, width), jnp.float32),
          pltpu.SemaphoreType.DMA,
      ],
  )
  def gather_kernel(table_hbm, idx_hbm, out_hbm, idx_v, rows_v, sem):
    wid = lax.axis_index("s") * 2 + lax.axis_index("c")

    def body(j, carry):
      base = wid * b_per_w + j * ch
      pltpu.sync_copy(idx_hbm.at[pl.ds(base, ch)], idx_v)
      pltpu.async_copy(table_hbm.at[idx_v], rows_v, sem).wait()
      pltpu.sync_copy(rows_v, out_hbm.at[pl.ds(base, ch)])
      return carry

    lax.fori_loop(0, nchunk, body, 0)

  return gather_kernel(table, idx)


def _mlp_body(instr_ref, flat_ref, conf_ref,
              w1a_ref, w1b_ref, b1_ref, w2_ref, b2_ref,
              wsa_ref, wsb_ref, wsc_ref, bs1_ref,
              ws2_ref, bs2_ref, ws3_ref, bs3_ref,
              wiv_ref, biv_ref,
              out_ref, pt_ref, iv_ref):
  instr = instr_ref[...]
  flat = flat_ref[...]
  mm = lambda a, b: jnp.dot(a, b, preferred_element_type=jnp.float32)
  h = jnp.maximum(mm(instr, w1a_ref[...]) + mm(flat, w1b_ref[...])
                  + b1_ref[...], 0.0)
  pt = mm(h, w2_ref[...]) + b2_ref[...]
  h2 = jnp.maximum(mm(pt, wsa_ref[...]) + mm(conf_ref[...], wsb_ref[...])
                   + mm(flat, wsc_ref[...]) + bs1_ref[...], 0.0)
  h3 = jnp.maximum(mm(h2, ws2_ref[...]) + bs2_ref[...], 0.0)
  out_ref[...] = mm(h3, ws3_ref[...]) + bs3_ref[...]
  pt_ref[...] = pt
  iv_ref[...] = mm(instr, wiv_ref[...]) + biv_ref[...]


def _mlp(instruments, flat, confounders, weights):
  b = flat.shape[0]
  bt = min(512, b)
  fixed = lambda a: pl.BlockSpec(a.shape, lambda i: (0,) * a.ndim)
  row = lambda a: pl.BlockSpec((bt, a.shape[1]), lambda i: (i, 0))

  return pl.pallas_call(
      _mlp_body,
      grid=(b // bt,),
      in_specs=[row(instruments), row(flat), row(confounders)]
      + [fixed(w) for w in weights],
      out_specs=[
          pl.BlockSpec((bt, 1), lambda i: (i, 0)),
          pl.BlockSpec((bt, 2), lambda i: (i, 0)),
          pl.BlockSpec((bt, 2), lambda i: (i, 0)),
      ],
      out_shape=[
          jax.ShapeDtypeStruct((b, 1), jnp.float32),
          jax.ShapeDtypeStruct((b, 2), jnp.float32),
          jax.ShapeDtypeStruct((b, 2), jnp.float32),
      ],
  )(instruments, flat, confounders, *weights)


def kernel(patient, treatment, confounders, corpus_embeddings,
           W_pe, b_pe, W_fs1, b_fs1, W_fs2, b_fs2,
           W_ss1, b_ss1, W_ss2, b_ss2, W_ss3, b_ss3,
           W_iv, b_iv, instruments):
  b = patient.shape[0]
  ndocs, emb = corpus_embeddings.shape
  k = _TOPK
  instr_d = instruments.shape[1]
  conf_d = confounders.shape[1]

  pe = patient @ W_pe.T + b_pe
  pe = pe / jnp.clip(jnp.sqrt(jnp.sum(pe * pe, axis=1, keepdims=True)), 1e-12)
  ndocs_pad = (ndocs + _DT - 1) // _DT * _DT
  cpad = jnp.pad(corpus_embeddings, ((0, ndocs_pad - ndocs), (0, 0)))
  cn = cpad / jnp.clip(
      jnp.sqrt(jnp.sum(cpad * cpad, axis=1, keepdims=True)), 1e-12)

  scores, cm, nchunk = _similarity_spill(pe, cn)
  rows, base = _top_chunks(cm, ndocs, nchunk)
  cand = _sc_gather(scores.reshape(nchunk * b, _CHUNK), rows.reshape(-1))
  scores8, idx8 = _top_docs(cand.reshape(b, k * _CHUNK), base, ndocs)

  flat = _sc_gather(corpus_embeddings, idx8.reshape(-1)).reshape(b, k * emb)

  weights = (
      W_fs1[:, :instr_d].T, W_fs1[:, instr_d:].T, b_fs1.reshape(1, -1),
      W_fs2.T, b_fs2.reshape(1, -1),
      W_ss1[:, :2].T, W_ss1[:, 2:2 + conf_d].T, W_ss1[:, 2 + conf_d:].T,
      b_ss1.reshape(1, -1),
      W_ss2.T, b_ss2.reshape(1, -1), W_ss3.T, b_ss3.reshape(1, -1),
      W_iv.T, b_iv.reshape(1, -1),
  )
  outcome, pt, iv = _mlp(instruments, flat, confounders, weights)

  return (outcome, scores8, idx8, pt, instruments, iv)


# speedup vs baseline: 10.5805x; 1.3466x over previous
import functools

import jax
import jax.numpy as jnp
from jax import lax
from jax.experimental import pallas as pl
from jax.experimental.pallas import tpu as pltpu
from jax.experimental.pallas import tpu_sc as plsc

_TOPK = 8
_NEG = -1e30
_BIGF = 1e9
_DT = 1024
_CHUNK = 256
_CPT = _DT // _CHUNK


def _stage1_body(qt, pe_ref, corpus_ref, scores_ref, cm_ref):
  d = pl.program_id(0)
  q = pl.program_id(1)
  w = cm_ref.shape[1]
  qs = pl.ds(q * qt, qt)

  s = lax.dot_general(pe_ref[qs, :], corpus_ref[...], (((1,), (1,)), ((), ())),
                      preferred_element_type=jnp.float32)
  for j in range(_CPT):
    scores_ref[j] = s[:, j * _CHUNK:(j + 1) * _CHUNK]

  lane = lax.broadcasted_iota(jnp.int32, (qt, w), 1)
  prev = jnp.where(d == 0, jnp.full((qt, w), _NEG, jnp.float32), cm_ref[qs, :])
  for j in range(_CPT):
    cj = jnp.max(s[:, j * _CHUNK:(j + 1) * _CHUNK], axis=1, keepdims=True)
    prev = jnp.where(lane == d * _CPT + j, cj, prev)
  cm_ref[qs, :] = prev


def _similarity_spill(pe_norm, corpus_pad):
  b, emb = pe_norm.shape
  ndocs_pad = corpus_pad.shape[0]
  qt = min(512, b)
  nchunk = ndocs_pad // _CHUNK
  w = (nchunk + 127) // 128 * 128
  grid = (ndocs_pad // _DT, b // qt)

  scores, cm = pl.pallas_call(
      functools.partial(_stage1_body, qt),
      grid=grid,
      in_specs=[
          pl.BlockSpec((b, emb), lambda d, q: (0, 0)),
          pl.BlockSpec((_DT, emb), lambda d, q: (d, 0)),
      ],
      out_specs=[
          pl.BlockSpec((_CPT, qt, _CHUNK), lambda d, q: (d, q, 0)),
          pl.BlockSpec((b, w), lambda d, q: (0, 0)),
      ],
      out_shape=[
          jax.ShapeDtypeStruct((nchunk, b, _CHUNK), jnp.float32),
          jax.ShapeDtypeStruct((b, w), jnp.float32),
      ],
      compiler_params=pltpu.CompilerParams(
          dimension_semantics=("arbitrary", "arbitrary")),
  )(pe_norm, corpus_pad)
  return scores, cm, nchunk


def _phaseb_body(ndocs, nchunk, cm_ref, rows_ref, base_ref):
  q = pl.program_id(0)
  bq, w = cm_ref.shape
  fp = -(-ndocs // _CHUNK)
  lane = lax.broadcasted_iota(jnp.int32, (bq, w), 1).astype(jnp.float32)
  cm = jnp.where(lane < fp, cm_ref[...], _NEG)
  rowi = (lax.broadcasted_iota(jnp.int32, (bq, _TOPK), 0).astype(jnp.float32)
          + q * bq)

  cids, vals = [], []
  for _ in range(_TOPK):
    m = jnp.max(cm, axis=1, keepdims=True)
    eq = cm == m
    cid = jnp.min(jnp.where(eq, lane, _BIGF), axis=1, keepdims=True)
    cids.append(cid)
    cm = jnp.where(lane == cid, _NEG, cm)
  cid8 = jnp.concatenate(cids, axis=1)
  rows_ref[...] = (cid8 * (bq * pl.num_programs(0)) + rowi).astype(jnp.int32)
  base_ref[...] = (cid8 * _CHUNK).astype(jnp.int32)


def _top_chunks(cm, ndocs, nchunk):
  b, w = cm.shape
  qt = min(512, b)
  return pl.pallas_call(
      functools.partial(_phaseb_body, ndocs, nchunk),
      grid=(b // qt,),
      in_specs=[pl.BlockSpec((qt, w), lambda q: (q, 0))],
      out_specs=[
          pl.BlockSpec((qt, _TOPK), lambda q: (q, 0)),
          pl.BlockSpec((qt, _TOPK), lambda q: (q, 0)),
      ],
      out_shape=[
          jax.ShapeDtypeStruct((b, _TOPK), jnp.int32),
          jax.ShapeDtypeStruct((b, _TOPK), jnp.int32),
      ],
  )(cm)


def _phased_body(ndocs, g_ref, base_ref, s8_ref, i8_ref):
  bq = g_ref.shape[0]
  kw = _TOPK * _CHUNK
  g = g_ref[...]
  basef = base_ref[...].astype(jnp.float32)
  offs = jnp.astype(
      lax.broadcasted_iota(jnp.int32, (bq, kw), 1) & (_CHUNK - 1),
      jnp.float32)
  baseexp = jnp.concatenate(
      [jnp.broadcast_to(basef[:, j:j + 1], (bq, _CHUNK))
       for j in range(_TOPK)], axis=1)
  gidx = baseexp + offs
  g = jnp.where(gidx < ndocs, g, _NEG)

  ss, ii = [], []
  for _ in range(_TOPK):
    m = jnp.max(g, axis=1, keepdims=True)
    eq = g == m
    gi = jnp.min(jnp.where(eq, gidx, _BIGF), axis=1, keepdims=True)
    ss.append(m)
    ii.append(gi)
    g = jnp.where(gidx == gi, _NEG, g)
  s8_ref[...] = jnp.concatenate(ss, axis=1)
  i8_ref[...] = jnp.concatenate(ii, axis=1).astype(jnp.int32)


def _top_docs(gathered, base, ndocs):
  b = gathered.shape[0]
  qt = min(512, b)
  kw = _TOPK * _CHUNK
  return pl.pallas_call(
      functools.partial(_phased_body, ndocs),
      grid=(b // qt,),
      in_specs=[
          pl.BlockSpec((qt, kw), lambda q: (q, 0)),
          pl.BlockSpec((qt, _TOPK), lambda q: (q, 0)),
      ],
      out_specs=[
          pl.BlockSpec((qt, _TOPK), lambda q: (q, 0)),
          pl.BlockSpec((qt, _TOPK), lambda q: (q, 0)),
      ],
      out_shape=[
          jax.ShapeDtypeStruct((b, _TOPK), jnp.float32),
          jax.ShapeDtypeStruct((b, _TOPK), jnp.int32),
      ],
  )(gathered, base)


def _sc_gather(table, idx):
  nrows, width = table.shape
  total = idx.shape[0]
  nw = 32
  b_per_w = total // nw
  ch = 128 if b_per_w % 128 == 0 else b_per_w
  nchunk = b_per_w // ch
  mesh = plsc.VectorSubcoreMesh(core_axis_name="c", subcore_axis_name="s")

  @functools.partial(
      pl.kernel,
      out_type=jax.ShapeDtypeStruct((total, width), jnp.float32),
      mesh=mesh,
      scratch_types=[
          pltpu.VMEM((ch,), jnp.int32),
          pltpu.VMEM((ch, width), jnp.float32),
          pltpu.SemaphoreType.DMA,
      ],
  )
  def gather_kernel(table_hbm, idx_hbm, out_hbm, idx_v, rows_v, sem):
    wid = lax.axis_index("s") * 2 + lax.axis_index("c")

    def body(j, carry):
      base = wid * b_per_w + j * ch
      pltpu.sync_copy(idx_hbm.at[pl.ds(base, ch)], idx_v)
      pltpu.async_copy(table_hbm.at[idx_v], rows_v, sem).wait()
      pltpu.sync_copy(rows_v, out_hbm.at[pl.ds(base, ch)])
      return carry

    lax.fori_loop(0, nchunk, body, 0)

  return gather_kernel(table, idx)


def _mlp_body(instr_ref, flat_ref, conf_ref,
              w1a_ref, w1b_ref, b1_ref, w2_ref, b2_ref,
              wsa_ref, wsb_ref, wsc_ref, bs1_ref,
              ws2_ref, bs2_ref, ws3_ref, bs3_ref,
              wiv_ref, biv_ref,
              out_ref, pt_ref, iv_ref):
  instr = instr_ref[...]
  flat = flat_ref[...]
  mm = lambda a, b: jnp.dot(a, b, preferred_element_type=jnp.float32)
  h = jnp.maximum(mm(instr, w1a_ref[...]) + mm(flat, w1b_ref[...])
                  + b1_ref[...], 0.0)
  pt = mm(h, w2_ref[...]) + b2_ref[...]
  h2 = jnp.maximum(mm(pt, wsa_ref[...]) + mm(conf_ref[...], wsb_ref[...])
                   + mm(flat, wsc_ref[...]) + bs1_ref[...], 0.0)
  h3 = jnp.maximum(mm(h2, ws2_ref[...]) + bs2_ref[...], 0.0)
  out_ref[...] = mm(h3, ws3_ref[...]) + bs3_ref[...]
  pt_ref[...] = pt
  iv_ref[...] = mm(instr, wiv_ref[...]) + biv_ref[...]


def _mlp(instruments, flat, confounders, weights):
  b = flat.shape[0]
  bt = min(512, b)
  fixed = lambda a: pl.BlockSpec(a.shape, lambda i: (0,) * a.ndim)
  row = lambda a: pl.BlockSpec((bt, a.shape[1]), lambda i: (i, 0))

  return pl.pallas_call(
      _mlp_body,
      grid=(b // bt,),
      in_specs=[row(instruments), row(flat), row(confounders)]
      + [fixed(w) for w in weights],
      out_specs=[
          pl.BlockSpec((bt, 1), lambda i: (i, 0)),
          pl.BlockSpec((bt, 2), lambda i: (i, 0)),
          pl.BlockSpec((bt, 2), lambda i: (i, 0)),
      ],
      out_shape=[
          jax.ShapeDtypeStruct((b, 1), jnp.float32),
          jax.ShapeDtypeStruct((b, 2), jnp.float32),
          jax.ShapeDtypeStruct((b, 2), jnp.float32),
      ],
  )(instruments, flat, confounders, *weights)


def kernel(patient, treatment, confounders, corpus_embeddings,
           W_pe, b_pe, W_fs1, b_fs1, W_fs2, b_fs2,
           W_ss1, b_ss1, W_ss2, b_ss2, W_ss3, b_ss3,
           W_iv, b_iv, instruments):
  b = patient.shape[0]
  ndocs, emb = corpus_embeddings.shape
  k = _TOPK
  instr_d = instruments.shape[1]
  conf_d = confounders.shape[1]

  pe = patient @ W_pe.T + b_pe
  pe = pe / jnp.clip(jnp.sqrt(jnp.sum(pe * pe, axis=1, keepdims=True)), 1e-12)
  ndocs_pad = (ndocs + _DT - 1) // _DT * _DT
  cpad = jnp.pad(corpus_embeddings, ((0, ndocs_pad - ndocs), (0, 0)))
  cn = cpad / jnp.clip(
      jnp.sqrt(jnp.sum(cpad * cpad, axis=1, keepdims=True)), 1e-12)

  scores, cm, nchunk = _similarity_spill(pe, cn)
  rows, base = _top_chunks(cm, ndocs, nchunk)
  cand = _sc_gather(scores.reshape(nchunk * b, _CHUNK), rows.reshape(-1))
  scores8, idx8 = _top_docs(cand.reshape(b, k * _CHUNK), base, ndocs)

  flat = _sc_gather(corpus_embeddings, idx8.reshape(-1)).reshape(b, k * emb)

  weights = (
      W_fs1[:, :instr_d].T, W_fs1[:, instr_d:].T, b_fs1.reshape(1, -1),
      W_fs2.T, b_fs2.reshape(1, -1),
      W_ss1[:, :2].T, W_ss1[:, 2:2 + conf_d].T, W_ss1[:, 2 + conf_d:].T,
      b_ss1.reshape(1, -1),
      W_ss2.T, b_ss2.reshape(1, -1), W_ss3.T, b_ss3.reshape(1, -1),
      W_iv.T, b_iv.reshape(1, -1),
  )
  outcome, pt, iv = _mlp(instruments, flat, confounders, weights)

  return (outcome, scores8, idx8, pt, instruments, iv)


# speedup vs baseline: 12.6567x; 1.1962x over previous
import functools

import jax
import jax.numpy as jnp
from jax import lax
from jax.experimental import pallas as pl
from jax.experimental.pallas import tpu as pltpu
from jax.experimental.pallas import tpu_sc as plsc

_TOPK = 8
_NEG = -1e30
_BIGF = 1e9
_DT = 2048
_CHUNK = 256
_CPT = _DT // _CHUNK


def _stage1_body(qt, pe_ref, corpus_ref, scores_ref, cm_ref):
  d = pl.program_id(0)
  q = pl.program_id(1)
  w = cm_ref.shape[1]
  qs = pl.ds(q * qt, qt)

  s = lax.dot_general(pe_ref[qs, :], corpus_ref[...], (((1,), (1,)), ((), ())),
                      preferred_element_type=jnp.float32)
  for j in range(_CPT):
    scores_ref[j] = s[:, j * _CHUNK:(j + 1) * _CHUNK]

  lane = lax.broadcasted_iota(jnp.int32, (qt, w), 1)
  prev = jnp.where(d == 0, jnp.full((qt, w), _NEG, jnp.float32), cm_ref[qs, :])
  for j in range(_CPT):
    cj = jnp.max(s[:, j * _CHUNK:(j + 1) * _CHUNK], axis=1, keepdims=True)
    prev = jnp.where(lane == d * _CPT + j, cj, prev)
  cm_ref[qs, :] = prev


def _similarity_spill(pe_norm, corpus_pad):
  b, emb = pe_norm.shape
  ndocs_pad = corpus_pad.shape[0]
  qt = min(512, b)
  nchunk = ndocs_pad // _CHUNK
  w = (nchunk + 127) // 128 * 128
  grid = (ndocs_pad // _DT, b // qt)

  scores, cm = pl.pallas_call(
      functools.partial(_stage1_body, qt),
      grid=grid,
      in_specs=[
          pl.BlockSpec((b, emb), lambda d, q: (0, 0)),
          pl.BlockSpec((_DT, emb), lambda d, q: (d, 0)),
      ],
      out_specs=[
          pl.BlockSpec((_CPT, qt, _CHUNK), lambda d, q: (d, q, 0)),
          pl.BlockSpec((b, w), lambda d, q: (0, 0)),
      ],
      out_shape=[
          jax.ShapeDtypeStruct((nchunk, b, _CHUNK), jnp.float32),
          jax.ShapeDtypeStruct((b, w), jnp.float32),
      ],
      compiler_params=pltpu.CompilerParams(
          dimension_semantics=("arbitrary", "arbitrary")),
  )(pe_norm, corpus_pad)
  return scores, cm, nchunk


def _phaseb_body(ndocs, nchunk, cm_ref, rows_ref, base_ref):
  q = pl.program_id(0)
  bq, w = cm_ref.shape
  fp = -(-ndocs // _CHUNK)
  lane = lax.broadcasted_iota(jnp.int32, (bq, w), 1).astype(jnp.float32)
  cm = jnp.where(lane < fp, cm_ref[...], _NEG)
  rowi = (lax.broadcasted_iota(jnp.int32, (bq, _TOPK), 0).astype(jnp.float32)
          + q * bq)

  cids, vals = [], []
  for _ in range(_TOPK):
    m = jnp.max(cm, axis=1, keepdims=True)
    eq = cm == m
    cid = jnp.min(jnp.where(eq, lane, _BIGF), axis=1, keepdims=True)
    cids.append(cid)
    cm = jnp.where(lane == cid, _NEG, cm)
  cid8 = jnp.concatenate(cids, axis=1)
  rows_ref[...] = (cid8 * (bq * pl.num_programs(0)) + rowi).astype(jnp.int32)
  base_ref[...] = (cid8 * _CHUNK).astype(jnp.int32)


def _top_chunks(cm, ndocs, nchunk):
  b, w = cm.shape
  qt = min(512, b)
  return pl.pallas_call(
      functools.partial(_phaseb_body, ndocs, nchunk),
      grid=(b // qt,),
      in_specs=[pl.BlockSpec((qt, w), lambda q: (q, 0))],
      out_specs=[
          pl.BlockSpec((qt, _TOPK), lambda q: (q, 0)),
          pl.BlockSpec((qt, _TOPK), lambda q: (q, 0)),
      ],
      out_shape=[
          jax.ShapeDtypeStruct((b, _TOPK), jnp.int32),
          jax.ShapeDtypeStruct((b, _TOPK), jnp.int32),
      ],
  )(cm)


def _phased_body(ndocs, g_ref, base_ref, s8_ref, i8_ref):
  bq = g_ref.shape[0]
  kw = _TOPK * _CHUNK
  g = g_ref[...]
  basef = base_ref[...].astype(jnp.float32)
  offs = jnp.astype(
      lax.broadcasted_iota(jnp.int32, (bq, kw), 1) & (_CHUNK - 1),
      jnp.float32)
  baseexp = jnp.concatenate(
      [jnp.broadcast_to(basef[:, j:j + 1], (bq, _CHUNK))
       for j in range(_TOPK)], axis=1)
  gidx = baseexp + offs
  g = jnp.where(gidx < ndocs, g, _NEG)

  ss, ii = [], []
  for _ in range(_TOPK):
    m = jnp.max(g, axis=1, keepdims=True)
    eq = g == m
    gi = jnp.min(jnp.where(eq, gidx, _BIGF), axis=1, keepdims=True)
    ss.append(m)
    ii.append(gi)
    g = jnp.where(gidx == gi, _NEG, g)
  s8_ref[...] = jnp.concatenate(ss, axis=1)
  i8_ref[...] = jnp.concatenate(ii, axis=1).astype(jnp.int32)


def _top_docs(gathered, base, ndocs):
  b = gathered.shape[0]
  qt = min(512, b)
  kw = _TOPK * _CHUNK
  return pl.pallas_call(
      functools.partial(_phased_body, ndocs),
      grid=(b // qt,),
      in_specs=[
          pl.BlockSpec((qt, kw), lambda q: (q, 0)),
          pl.BlockSpec((qt, _TOPK), lambda q: (q, 0)),
      ],
      out_specs=[
          pl.BlockSpec((qt, _TOPK), lambda q: (q, 0)),
          pl.BlockSpec((qt, _TOPK), lambda q: (q, 0)),
      ],
      out_shape=[
          jax.ShapeDtypeStruct((b, _TOPK), jnp.float32),
          jax.ShapeDtypeStruct((b, _TOPK), jnp.int32),
      ],
  )(gathered, base)


def _sc_gather(table, idx):
  nrows, width = table.shape
  total = idx.shape[0]
  nw = 32
  b_per_w = total // nw
  ch = 128 if b_per_w % 128 == 0 else b_per_w
  nchunk = b_per_w // ch
  mesh = plsc.VectorSubcoreMesh(core_axis_name="c", subcore_axis_name="s")

  @functools.partial(
      pl.kernel,
      out_type=jax.ShapeDtypeStruct((total, width), jnp.float32),
      mesh=mesh,
      scratch_types=[
          pltpu.VMEM((ch,), jnp.int32),
          pltpu.VMEM((ch, width), jnp.float32),
          pltpu.SemaphoreType.DMA,
      ],
  )
  def gather_kernel(table_hbm, idx_hbm, out_hbm, idx_v, rows_v, sem):
    wid = lax.axis_index("s") * 2 + lax.axis_index("c")

    def body(j, carry):
      base = wid * b_per_w + j * ch
      pltpu.sync_copy(idx_hbm.at[pl.ds(base, ch)], idx_v)
      pltpu.async_copy(table_hbm.at[idx_v], rows_v, sem).wait()
      pltpu.sync_copy(rows_v, out_hbm.at[pl.ds(base, ch)])
      return carry

    lax.fori_loop(0, nchunk, body, 0)

  return gather_kernel(table, idx)


def _mlp_body(instr_ref, flat_ref, conf_ref,
              w1a_ref, w1b_ref, b1_ref, w2_ref, b2_ref,
              wsa_ref, wsb_ref, wsc_ref, bs1_ref,
              ws2_ref, bs2_ref, ws3_ref, bs3_ref,
              wiv_ref, biv_ref,
              out_ref, pt_ref, iv_ref):
  instr = instr_ref[...]
  flat = flat_ref[...]
  mm = lambda a, b: jnp.dot(a, b, preferred_element_type=jnp.float32)
  h = jnp.maximum(mm(instr, w1a_ref[...]) + mm(flat, w1b_ref[...])
                  + b1_ref[...], 0.0)
  pt = mm(h, w2_ref[...]) + b2_ref[...]
  h2 = jnp.maximum(mm(pt, wsa_ref[...]) + mm(conf_ref[...], wsb_ref[...])
                   + mm(flat, wsc_ref[...]) + bs1_ref[...], 0.0)
  h3 = jnp.maximum(mm(h2, ws2_ref[...]) + bs2_ref[...], 0.0)
  out_ref[...] = mm(h3, ws3_ref[...]) + bs3_ref[...]
  pt_ref[...] = pt
  iv_ref[...] = mm(instr, wiv_ref[...]) + biv_ref[...]


def _mlp(instruments, flat, confounders, weights):
  b = flat.shape[0]
  bt = min(512, b)
  fixed = lambda a: pl.BlockSpec(a.shape, lambda i: (0,) * a.ndim)
  row = lambda a: pl.BlockSpec((bt, a.shape[1]), lambda i: (i, 0))

  return pl.pallas_call(
      _mlp_body,
      grid=(b // bt,),
      in_specs=[row(instruments), row(flat), row(confounders)]
      + [fixed(w) for w in weights],
      out_specs=[
          pl.BlockSpec((bt, 1), lambda i: (i, 0)),
          pl.BlockSpec((bt, 2), lambda i: (i, 0)),
          pl.BlockSpec((bt, 2), lambda i: (i, 0)),
      ],
      out_shape=[
          jax.ShapeDtypeStruct((b, 1), jnp.float32),
          jax.ShapeDtypeStruct((b, 2), jnp.float32),
          jax.ShapeDtypeStruct((b, 2), jnp.float32),
      ],
  )(instruments, flat, confounders, *weights)


def kernel(patient, treatment, confounders, corpus_embeddings,
           W_pe, b_pe, W_fs1, b_fs1, W_fs2, b_fs2,
           W_ss1, b_ss1, W_ss2, b_ss2, W_ss3, b_ss3,
           W_iv, b_iv, instruments):
  b = patient.shape[0]
  ndocs, emb = corpus_embeddings.shape
  k = _TOPK
  instr_d = instruments.shape[1]
  conf_d = confounders.shape[1]

  pe = patient @ W_pe.T + b_pe
  pe = pe / jnp.clip(jnp.sqrt(jnp.sum(pe * pe, axis=1, keepdims=True)), 1e-12)
  ndocs_pad = (ndocs + _DT - 1) // _DT * _DT
  cpad = jnp.pad(corpus_embeddings, ((0, ndocs_pad - ndocs), (0, 0)))
  cn = cpad / jnp.clip(
      jnp.sqrt(jnp.sum(cpad * cpad, axis=1, keepdims=True)), 1e-12)

  scores, cm, nchunk = _similarity_spill(pe, cn)
  rows, base = _top_chunks(cm, ndocs, nchunk)
  cand = _sc_gather(scores.reshape(nchunk * b, _CHUNK), rows.reshape(-1))
  scores8, idx8 = _top_docs(cand.reshape(b, k * _CHUNK), base, ndocs)

  flat = _sc_gather(corpus_embeddings, idx8.reshape(-1)).reshape(b, k * emb)

  weights = (
      W_fs1[:, :instr_d].T, W_fs1[:, instr_d:].T, b_fs1.reshape(1, -1),
      W_fs2.T, b_fs2.reshape(1, -1),
      W_ss1[:, :2].T, W_ss1[:, 2:2 + conf_d].T, W_ss1[:, 2 + conf_d:].T,
      b_ss1.reshape(1, -1),
      W_ss2.T, b_ss2.reshape(1, -1), W_ss3.T, b_ss3.reshape(1, -1),
      W_iv.T, b_iv.reshape(1, -1),
  )
  outcome, pt, iv = _mlp(instruments, flat, confounders, weights)

  return (outcome, scores8, idx8, pt, instruments, iv)


# speedup vs baseline: 13.8805x; 1.0967x over previous
import functools

import jax
import jax.numpy as jnp
from jax import lax
from jax.experimental import pallas as pl
from jax.experimental.pallas import tpu as pltpu
from jax.experimental.pallas import tpu_sc as plsc

_TOPK = 8
_NEG = -1e30
_BIGF = 1e9
_DT = 4096
_CHUNK = 256
_CPT = _DT // _CHUNK


def _stage1_body(qt, pe_ref, corpus_ref, scores_ref, cm_ref):
  d = pl.program_id(0)
  q = pl.program_id(1)
  w = cm_ref.shape[1]
  qs = pl.ds(q * qt, qt)

  s = lax.dot_general(pe_ref[qs, :], corpus_ref[...], (((1,), (1,)), ((), ())),
                      preferred_element_type=jnp.float32)
  for j in range(_CPT):
    scores_ref[j] = s[:, j * _CHUNK:(j + 1) * _CHUNK]

  lane = lax.broadcasted_iota(jnp.int32, (qt, w), 1)
  prev = jnp.where(d == 0, jnp.full((qt, w), _NEG, jnp.float32), cm_ref[qs, :])
  for j in range(_CPT):
    cj = jnp.max(s[:, j * _CHUNK:(j + 1) * _CHUNK], axis=1, keepdims=True)
    prev = jnp.where(lane == d * _CPT + j, cj, prev)
  cm_ref[qs, :] = prev


def _similarity_spill(pe_norm, corpus_pad):
  b, emb = pe_norm.shape
  ndocs_pad = corpus_pad.shape[0]
  qt = min(512, b)
  nchunk = ndocs_pad // _CHUNK
  w = (nchunk + 127) // 128 * 128
  grid = (ndocs_pad // _DT, b // qt)

  scores, cm = pl.pallas_call(
      functools.partial(_stage1_body, qt),
      grid=grid,
      in_specs=[
          pl.BlockSpec((b, emb), lambda d, q: (0, 0)),
          pl.BlockSpec((_DT, emb), lambda d, q: (d, 0)),
      ],
      out_specs=[
          pl.BlockSpec((_CPT, qt, _CHUNK), lambda d, q: (d, q, 0)),
          pl.BlockSpec((b, w), lambda d, q: (0, 0)),
      ],
      out_shape=[
          jax.ShapeDtypeStruct((nchunk, b, _CHUNK), jnp.float32),
          jax.ShapeDtypeStruct((b, w), jnp.float32),
      ],
      compiler_params=pltpu.CompilerParams(
          dimension_semantics=("arbitrary", "arbitrary")),
  )(pe_norm, corpus_pad)
  return scores, cm, nchunk


def _phaseb_body(ndocs, nchunk, cm_ref, rows_ref, base_ref):
  q = pl.program_id(0)
  bq, w = cm_ref.shape
  fp = -(-ndocs // _CHUNK)
  lane = lax.broadcasted_iota(jnp.int32, (bq, w), 1).astype(jnp.float32)
  cm = jnp.where(lane < fp, cm_ref[...], _NEG)
  rowi = (lax.broadcasted_iota(jnp.int32, (bq, _TOPK), 0).astype(jnp.float32)
          + q * bq)

  cids, vals = [], []
  for _ in range(_TOPK):
    m = jnp.max(cm, axis=1, keepdims=True)
    eq = cm == m
    cid = jnp.min(jnp.where(eq, lane, _BIGF), axis=1, keepdims=True)
    cids.append(cid)
    cm = jnp.where(lane == cid, _NEG, cm)
  cid8 = jnp.concatenate(cids, axis=1)
  rows_ref[...] = (cid8 * (bq * pl.num_programs(0)) + rowi).astype(jnp.int32)
  base_ref[...] = (cid8 * _CHUNK).astype(jnp.int32)


def _top_chunks(cm, ndocs, nchunk):
  b, w = cm.shape
  qt = min(512, b)
  return pl.pallas_call(
      functools.partial(_phaseb_body, ndocs, nchunk),
      grid=(b // qt,),
      in_specs=[pl.BlockSpec((qt, w), lambda q: (q, 0))],
      out_specs=[
          pl.BlockSpec((qt, _TOPK), lambda q: (q, 0)),
          pl.BlockSpec((qt, _TOPK), lambda q: (q, 0)),
      ],
      out_shape=[
          jax.ShapeDtypeStruct((b, _TOPK), jnp.int32),
          jax.ShapeDtypeStruct((b, _TOPK), jnp.int32),
      ],
  )(cm)


def _phased_body(ndocs, g_ref, base_ref, s8_ref, i8_ref):
  bq = g_ref.shape[0]
  kw = _TOPK * _CHUNK
  g = g_ref[...]
  basef = base_ref[...].astype(jnp.float32)
  offs = jnp.astype(
      lax.broadcasted_iota(jnp.int32, (bq, kw), 1) & (_CHUNK - 1),
      jnp.float32)
  baseexp = jnp.concatenate(
      [jnp.broadcast_to(basef[:, j:j + 1], (bq, _CHUNK))
       for j in range(_TOPK)], axis=1)
  gidx = baseexp + offs
  g = jnp.where(gidx < ndocs, g, _NEG)

  ss, ii = [], []
  for _ in range(_TOPK):
    m = jnp.max(g, axis=1, keepdims=True)
    eq = g == m
    gi = jnp.min(jnp.where(eq, gidx, _BIGF), axis=1, keepdims=True)
    ss.append(m)
    ii.append(gi)
    g = jnp.where(gidx == gi, _NEG, g)
  s8_ref[...] = jnp.concatenate(ss, axis=1)
  i8_ref[...] = jnp.concatenate(ii, axis=1).astype(jnp.int32)


def _top_docs(gathered, base, ndocs):
  b = gathered.shape[0]
  qt = min(512, b)
  kw = _TOPK * _CHUNK
  return pl.pallas_call(
      functools.partial(_phased_body, ndocs),
      grid=(b // qt,),
      in_specs=[
          pl.BlockSpec((qt, kw), lambda q: (q, 0)),
          pl.BlockSpec((qt, _TOPK), lambda q: (q, 0)),
      ],
      out_specs=[
          pl.BlockSpec((qt, _TOPK), lambda q: (q, 0)),
          pl.BlockSpec((qt, _TOPK), lambda q: (q, 0)),
      ],
      out_shape=[
          jax.ShapeDtypeStruct((b, _TOPK), jnp.float32),
          jax.ShapeDtypeStruct((b, _TOPK), jnp.int32),
      ],
  )(gathered, base)


def _sc_gather(table, idx):
  nrows, width = table.shape
  total = idx.shape[0]
  nw = 32
  b_per_w = total // nw
  ch = 128 if b_per_w % 128 == 0 else b_per_w
  nchunk = b_per_w // ch
  mesh = plsc.VectorSubcoreMesh(core_axis_name="c", subcore_axis_name="s")

  @functools.partial(
      pl.kernel,
      out_type=jax.ShapeDtypeStruct((total, width), jnp.float32),
      mesh=mesh,
      scratch_types=[
          pltpu.VMEM((ch,), jnp.int32),
          pltpu.VMEM((ch, width), jnp.float32),
          pltpu.SemaphoreType.DMA,
      ],
  )
  def gather_kernel(table_hbm, idx_hbm, out_hbm, idx_v, rows_v, sem):
    wid = lax.axis_index("s") * 2 + lax.axis_index("c")

    def body(j, carry):
      base = wid * b_per_w + j * ch
      pltpu.sync_copy(idx_hbm.at[pl.ds(base, ch)], idx_v)
      pltpu.async_copy(table_hbm.at[idx_v], rows_v, sem).wait()
      pltpu.sync_copy(rows_v, out_hbm.at[pl.ds(base, ch)])
      return carry

    lax.fori_loop(0, nchunk, body, 0)

  return gather_kernel(table, idx)


def _mlp_body(instr_ref, flat_ref, conf_ref,
              w1a_ref, w1b_ref, b1_ref, w2_ref, b2_ref,
              wsa_ref, wsb_ref, wsc_ref, bs1_ref,
              ws2_ref, bs2_ref, ws3_ref, bs3_ref,
              wiv_ref, biv_ref,
              out_ref, pt_ref, iv_ref):
  instr = instr_ref[...]
  flat = flat_ref[...]
  mm = lambda a, b: jnp.dot(a, b, preferred_element_type=jnp.float32)
  h = jnp.maximum(mm(instr, w1a_ref[...]) + mm(flat, w1b_ref[...])
                  + b1_ref[...], 0.0)
  pt = mm(h, w2_ref[...]) + b2_ref[...]
  h2 = jnp.maximum(mm(pt, wsa_ref[...]) + mm(conf_ref[...], wsb_ref[...])
                   + mm(flat, wsc_ref[...]) + bs1_ref[...], 0.0)
  h3 = jnp.maximum(mm(h2, ws2_ref[...]) + bs2_ref[...], 0.0)
  out_ref[...] = mm(h3, ws3_ref[...]) + bs3_ref[...]
  pt_ref[...] = pt
  iv_ref[...] = mm(instr, wiv_ref[...]) + biv_ref[...]


def _mlp(instruments, flat, confounders, weights):
  b = flat.shape[0]
  bt = min(512, b)
  fixed = lambda a: pl.BlockSpec(a.shape, lambda i: (0,) * a.ndim)
  row = lambda a: pl.BlockSpec((bt, a.shape[1]), lambda i: (i, 0))

  return pl.pallas_call(
      _mlp_body,
      grid=(b // bt,),
      in_specs=[row(instruments), row(flat), row(confounders)]
      + [fixed(w) for w in weights],
      out_specs=[
          pl.BlockSpec((bt, 1), lambda i: (i, 0)),
          pl.BlockSpec((bt, 2), lambda i: (i, 0)),
          pl.BlockSpec((bt, 2), lambda i: (i, 0)),
      ],
      out_shape=[
          jax.ShapeDtypeStruct((b, 1), jnp.float32),
          jax.ShapeDtypeStruct((b, 2), jnp.float32),
          jax.ShapeDtypeStruct((b, 2), jnp.float32),
      ],
  )(instruments, flat, confounders, *weights)


def kernel(patient, treatment, confounders, corpus_embeddings,
           W_pe, b_pe, W_fs1, b_fs1, W_fs2, b_fs2,
           W_ss1, b_ss1, W_ss2, b_ss2, W_ss3, b_ss3,
           W_iv, b_iv, instruments):
  b = patient.shape[0]
  ndocs, emb = corpus_embeddings.shape
  k = _TOPK
  instr_d = instruments.shape[1]
  conf_d = confounders.shape[1]

  pe = patient @ W_pe.T + b_pe
  pe = pe / jnp.clip(jnp.sqrt(jnp.sum(pe * pe, axis=1, keepdims=True)), 1e-12)
  ndocs_pad = (ndocs + _DT - 1) // _DT * _DT
  cpad = jnp.pad(corpus_embeddings, ((0, ndocs_pad - ndocs), (0, 0)))
  cn = cpad / jnp.clip(
      jnp.sqrt(jnp.sum(cpad * cpad, axis=1, keepdims=True)), 1e-12)

  scores, cm, nchunk = _similarity_spill(pe, cn)
  rows, base = _top_chunks(cm, ndocs, nchunk)
  cand = _sc_gather(scores.reshape(nchunk * b, _CHUNK), rows.reshape(-1))
  scores8, idx8 = _top_docs(cand.reshape(b, k * _CHUNK), base, ndocs)

  flat = _sc_gather(corpus_embeddings, idx8.reshape(-1)).reshape(b, k * emb)

  weights = (
      W_fs1[:, :instr_d].T, W_fs1[:, instr_d:].T, b_fs1.reshape(1, -1),
      W_fs2.T, b_fs2.reshape(1, -1),
      W_ss1[:, :2].T, W_ss1[:, 2:2 + conf_d].T, W_ss1[:, 2 + conf_d:].T,
      b_ss1.reshape(1, -1),
      W_ss2.T, b_ss2.reshape(1, -1), W_ss3.T, b_ss3.reshape(1, -1),
      W_iv.T, b_iv.reshape(1, -1),
  )
  outcome, pt, iv = _mlp(instruments, flat, confounders, weights)

  return (outcome, scores8, idx8, pt, instruments, iv)
